# baseline (speedup 1.0000x reference)
import functools

import jax
import jax.numpy as jnp
from jax import lax
from jax.experimental import pallas as pl
from jax.experimental.pallas import tpu as pltpu
from jax.experimental.pallas import tpu_sc as plsc

N, D, DEG, H = 10000, 128, 32, 256

NC, NS, L = 2, 16, 16
NW = NC * NS
PER_W = 320
N_PAD = NW * PER_W
GN = 4
GI = GN * DEG
NG = PER_W // GN
LV = D // L


NBUF = 4


def _sc_pool_body(x_hbm, nbr_hbm, out_hbm, idx_v, rows0, rows1, rows2,
                  rows3, outbuf, sem0, sem1, sem2, sem3):
    cid = lax.axis_index("c")
    sid = lax.axis_index("s")
    wid = sid * NC + cid
    base = wid * PER_W
    bufs = (rows0, rows1, rows2, rows3)
    sems = (sem0, sem1, sem2, sem3)

    pltpu.sync_copy(nbr_hbm.at[pl.ds(base * DEG, PER_W * DEG)], idx_v)

    def fire(g, b):
        pltpu.async_copy(x_hbm.at[idx_v.at[pl.ds(g * GI, GI)]],
                         bufs[b], sems[b])

    def reduce_group(g, b):
        rows = bufs[b]

        def node(n, carry):
            row = g * GN + n
            for ci in range(LV):
                v = rows[n * DEG, pl.ds(ci * L, L)]
                for j in range(1, DEG):
                    v = jnp.maximum(v, rows[n * DEG + j, pl.ds(ci * L, L)])
                outbuf[row, pl.ds(ci * L, L)] = v
            return carry

        lax.fori_loop(0, GN, node, 0, unroll=False)

    for b in range(NBUF - 1):
        fire(b, b)

    def ring(i, carry):
        for b in range(NBUF):
            g = i * NBUF + b
            nxt = g + NBUF - 1

            @pl.when(nxt < NG)
            def _():
                fire(nxt, (b + NBUF - 1) % NBUF)

            pltpu.make_async_copy(
                x_hbm.at[idx_v.at[pl.ds(g * GI, GI)]], bufs[b], sems[b]
            ).wait()
            reduce_group(g, b)
        return carry

    lax.fori_loop(0, NG // NBUF, ring, 0, unroll=False)

    pltpu.sync_copy(outbuf, out_hbm.at[pl.ds(base, PER_W)])


@functools.cache
def _sc_pool():
    return pl.kernel(
        _sc_pool_body,
        out_type=jax.ShapeDtypeStruct((N_PAD, D), jnp.float32),
        mesh=plsc.VectorSubcoreMesh(
            core_axis_name="c", subcore_axis_name="s",
            num_cores=NC, num_subcores=NS,
        ),
        scratch_types=[
            pltpu.VMEM((PER_W * DEG,), jnp.int32),
            pltpu.VMEM((GI, D), jnp.float32),
            pltpu.VMEM((GI, D), jnp.float32),
            pltpu.VMEM((GI, D), jnp.float32),
            pltpu.VMEM((GI, D), jnp.float32),
            pltpu.VMEM((PER_W, D), jnp.float32),
            pltpu.SemaphoreType.DMA,
            pltpu.SemaphoreType.DMA,
            pltpu.SemaphoreType.DMA,
            pltpu.SemaphoreType.DMA,
        ],
    )


BM = 1000
NB = N // BM


def _mlp_body(pooled, x, W1, b1, W2, b2, h2_out, stats_out):
    i = pl.program_id(0)
    h = pooled[...] + x[...]
    h1 = lax.dot_general(h, W1[...], (((1,), (0,)), ((), ())),
                         preferred_element_type=jnp.float32)
    h1 = jnp.maximum(h1 + b1[...], 0.0)
    h2 = lax.dot_general(h1, W2[...], (((1,), (0,)), ((), ())),
                         preferred_element_type=jnp.float32)
    h2 = jnp.maximum(h2 + b2[...], 0.0)
    h2_out[...] = h2
    ps = jnp.concatenate(
        [jnp.sum(h2, axis=0, keepdims=True),
         jnp.sum(h2 * h2, axis=0, keepdims=True)], axis=0)

    @pl.when(i == 0)
    def _():
        stats_out[...] = ps

    @pl.when(i > 0)
    def _():
        stats_out[...] = stats_out[...] + ps


def _norm_body(h2, stats, gamma, beta, out):
    s = stats[...]
    mean = s[0:1, :] * (1.0 / N)
    var = s[1:2, :] * (1.0 / N) - mean * mean
    scale = lax.rsqrt(var + 1e-5) * gamma[...]
    out[...] = (h2[...] - mean) * scale + beta[...]


def kernel(x, padded_neighbors, W1, b1, W2, b2, gamma, beta):
    nbrs = jnp.concatenate(
        [padded_neighbors,
         jnp.zeros((N_PAD - N, DEG), jnp.int32)], axis=0).reshape(-1)

    pooled = _sc_pool()(x, nbrs)[:N]

    h2, stats = pl.pallas_call(
        _mlp_body,
        grid=(NB,),
        in_specs=[
            pl.BlockSpec((BM, D), lambda i: (i, 0)),
            pl.BlockSpec((BM, D), lambda i: (i, 0)),
            pl.BlockSpec((D, H), lambda i: (0, 0)),
            pl.BlockSpec((1, H), lambda i: (0, 0)),
            pl.BlockSpec((H, D), lambda i: (0, 0)),
            pl.BlockSpec((1, D), lambda i: (0, 0)),
        ],
        out_specs=[
            pl.BlockSpec((BM, D), lambda i: (i, 0)),
            pl.BlockSpec((2, D), lambda i: (0, 0)),
        ],
        out_shape=[
            jax.ShapeDtypeStruct((N, D), jnp.float32),
            jax.ShapeDtypeStruct((2, D), jnp.float32),
        ],
    )(pooled, x, W1, b1.reshape(1, H), W2, b2.reshape(1, D))

    out = pl.pallas_call(
        _norm_body,
        grid=(NB,),
        in_specs=[
            pl.BlockSpec((BM, D), lambda i: (i, 0)),
            pl.BlockSpec((2, D), lambda i: (0, 0)),
            pl.BlockSpec((1, D), lambda i: (0, 0)),
            pl.BlockSpec((1, D), lambda i: (0, 0)),
        ],
        out_specs=pl.BlockSpec((BM, D), lambda i: (i, 0)),
        out_shape=jax.ShapeDtypeStruct((N, D), jnp.float32),
    )(h2, stats, gamma.reshape(1, D), beta.reshape(1, D))

    return out


# speedup vs baseline: 1.1235x; 1.1235x over previous
import functools

import jax
import jax.numpy as jnp
from jax import lax
from jax.experimental import pallas as pl
from jax.experimental.pallas import tpu as pltpu
from jax.experimental.pallas import tpu_sc as plsc

N, D, DEG, H = 10000, 128, 32, 256

NC, NS, L = 2, 16, 16
N_PAD = 10240
GN = 4
GI = GN * DEG
LV = D // L
NBUF = 4

FAST_NODES = 512
SLOW_NODES = 128
CHUNK = 128
CG = CHUNK // GN
MAXG = FAST_NODES // GN
NBR_PAD = 10624


def _sc_pool_body(x_hbm, nbr_hbm, out_hbm, idx_v, rows0, rows1, rows2,
                  rows3, outbuf, sem0, sem1, sem2, sem3):
    cid = lax.axis_index("c")
    sid = lax.axis_index("s")
    fast = cid == 1
    node_base = jnp.where(fast, sid * FAST_NODES,
                          16 * FAST_NODES + sid * SLOW_NODES)
    nchunks = jnp.where(fast, FAST_NODES // CHUNK, SLOW_NODES // CHUNK)
    bufs = (rows0, rows1, rows2, rows3)
    sems = (sem0, sem1, sem2, sem3)

    pltpu.sync_copy(nbr_hbm.at[pl.ds(node_base * DEG, FAST_NODES * DEG)],
                    idx_v)

    def fire(g, b):
        pltpu.async_copy(x_hbm.at[idx_v.at[pl.ds(g * GI, GI)]],
                         bufs[b], sems[b])

    def reduce_group(g, gc, b):
        rows = bufs[b]

        def node(n, carry):
            row = gc * GN + n
            for ci in range(LV):
                v = rows[n * DEG, pl.ds(ci * L, L)]
                for j in range(1, DEG):
                    v = jnp.maximum(v, rows[n * DEG + j, pl.ds(ci * L, L)])
                outbuf[row, pl.ds(ci * L, L)] = v
            return carry

        lax.fori_loop(0, GN, node, 0, unroll=False)

    def chunk_body(c, carry):
        g0 = c * CG
        for b in range(NBUF - 1):
            fire(g0 + b, b)

        def ring(i, carry2):
            for b in range(NBUF):
                gc = i * NBUF + b
                nxt = gc + NBUF - 1

                @pl.when(nxt < CG)
                def _():
                    fire(g0 + nxt, (b + NBUF - 1) % NBUF)

                pltpu.make_async_copy(
                    x_hbm.at[idx_v.at[pl.ds((g0 + gc) * GI, GI)]],
                    bufs[b], sems[b]
                ).wait()
                reduce_group(g0 + gc, gc, b)
            return carry2

        lax.fori_loop(0, CG // NBUF, ring, 0, unroll=False)

        pltpu.sync_copy(outbuf, out_hbm.at[pl.ds(node_base + c * CHUNK,
                                                 CHUNK)])
        return carry

    lax.fori_loop(0, nchunks, chunk_body, 0, unroll=False)


@functools.cache
def _sc_pool():
    return pl.kernel(
        _sc_pool_body,
        out_type=jax.ShapeDtypeStruct((N_PAD, D), jnp.float32),
        mesh=plsc.VectorSubcoreMesh(
            core_axis_name="c", subcore_axis_name="s",
            num_cores=NC, num_subcores=NS,
        ),
        scratch_types=[
            pltpu.VMEM((FAST_NODES * DEG,), jnp.int32),
            pltpu.VMEM((GI, D), jnp.float32),
            pltpu.VMEM((GI, D), jnp.float32),
            pltpu.VMEM((GI, D), jnp.float32),
            pltpu.VMEM((GI, D), jnp.float32),
            pltpu.VMEM((CHUNK, D), jnp.float32),
            pltpu.SemaphoreType.DMA,
            pltpu.SemaphoreType.DMA,
            pltpu.SemaphoreType.DMA,
            pltpu.SemaphoreType.DMA,
        ],
    )


BM = 1000
NB = N // BM


def _mlp_body(pooled, x, W1, b1, W2, b2, h2_out, stats_out):
    i = pl.program_id(0)
    h = pooled[...] + x[...]
    h1 = lax.dot_general(h, W1[...], (((1,), (0,)), ((), ())),
                         preferred_element_type=jnp.float32)
    h1 = jnp.maximum(h1 + b1[...], 0.0)
    h2 = lax.dot_general(h1, W2[...], (((1,), (0,)), ((), ())),
                         preferred_element_type=jnp.float32)
    h2 = jnp.maximum(h2 + b2[...], 0.0)
    h2_out[...] = h2
    ps = jnp.concatenate(
        [jnp.sum(h2, axis=0, keepdims=True),
         jnp.sum(h2 * h2, axis=0, keepdims=True)], axis=0)

    @pl.when(i == 0)
    def _():
        stats_out[...] = ps

    @pl.when(i > 0)
    def _():
        stats_out[...] = stats_out[...] + ps


def _norm_body(h2, stats, gamma, beta, out):
    s = stats[...]
    mean = s[0:1, :] * (1.0 / N)
    var = s[1:2, :] * (1.0 / N) - mean * mean
    scale = lax.rsqrt(var + 1e-5) * gamma[...]
    out[...] = (h2[...] - mean) * scale + beta[...]


def kernel(x, padded_neighbors, W1, b1, W2, b2, gamma, beta):
    nbrs = jnp.concatenate(
        [padded_neighbors,
         jnp.zeros((NBR_PAD - N, DEG), jnp.int32)], axis=0).reshape(-1)

    pooled = _sc_pool()(x, nbrs)[:N]

    h2, stats = pl.pallas_call(
        _mlp_body,
        grid=(NB,),
        in_specs=[
            pl.BlockSpec((BM, D), lambda i: (i, 0)),
            pl.BlockSpec((BM, D), lambda i: (i, 0)),
            pl.BlockSpec((D, H), lambda i: (0, 0)),
            pl.BlockSpec((1, H), lambda i: (0, 0)),
            pl.BlockSpec((H, D), lambda i: (0, 0)),
            pl.BlockSpec((1, D), lambda i: (0, 0)),
        ],
        out_specs=[
            pl.BlockSpec((BM, D), lambda i: (i, 0)),
            pl.BlockSpec((2, D), lambda i: (0, 0)),
        ],
        out_shape=[
            jax.ShapeDtypeStruct((N, D), jnp.float32),
            jax.ShapeDtypeStruct((2, D), jnp.float32),
        ],
    )(pooled, x, W1, b1.reshape(1, H), W2, b2.reshape(1, D))

    out = pl.pallas_call(
        _norm_body,
        grid=(NB,),
        in_specs=[
            pl.BlockSpec((BM, D), lambda i: (i, 0)),
            pl.BlockSpec((2, D), lambda i: (0, 0)),
            pl.BlockSpec((1, D), lambda i: (0, 0)),
            pl.BlockSpec((1, D), lambda i: (0, 0)),
        ],
        out_specs=pl.BlockSpec((BM, D), lambda i: (i, 0)),
        out_shape=jax.ShapeDtypeStruct((N, D), jnp.float32),
    )(h2, stats, gamma.reshape(1, D), beta.reshape(1, D))

    return out
